# baseline (speedup 1.0000x reference)
import functools
import jax
import jax.numpy as jnp
from jax.experimental import pallas as pl
from jax.experimental.pallas import tpu as pltpu

QP = 64
C = 64
NPARAM = 11


def _body(idx_ref, p_ref, x_ref, o_ref):
    del idx_ref
    p = p_ref[0]
    x = x_ref[...]

    def row(i):
        return p[i].reshape(1, C, 1, 1)

    y = x
    for layer in range(3):
        h, b, a = row(3 * layer), row(3 * layer + 1), row(3 * layer + 2)
        y = y * jax.nn.softplus(h) + b
        y = y + jnp.tanh(y) * jnp.tanh(a)
    y = y * jax.nn.softplus(row(9)) + row(10)
    o_ref[...] = jax.nn.sigmoid(y)


@jax.jit
def kernel(x, index, h1, b1, a1, h2, b2, a2, h3, b3, a3, h4, b4):
    B, Cx, H, W = x.shape
    table = jnp.stack(
        [t.reshape(QP, C) for t in (h1, b1, a1, h2, b2, a2, h3, b3, a3, h4, b4)],
        axis=1,
    )

    grid_spec = pltpu.PrefetchScalarGridSpec(
        num_scalar_prefetch=1,
        grid=(B,),
        in_specs=[
            pl.BlockSpec((1, NPARAM, C), lambda b, idx: (idx[b], 0, 0)),
            pl.BlockSpec((1, Cx, H, W), lambda b, idx: (b, 0, 0, 0)),
        ],
        out_specs=pl.BlockSpec((1, Cx, H, W), lambda b, idx: (b, 0, 0, 0)),
    )
    return pl.pallas_call(
        _body,
        grid_spec=grid_spec,
        out_shape=jax.ShapeDtypeStruct(x.shape, x.dtype),
    )(index, table, x)
